# baseline (speedup 1.0000x reference)
import jax
import jax.numpy as jnp
from jax import lax
from jax.experimental import pallas as pl
from jax.experimental.pallas import tpu as pltpu
from jax.experimental.pallas import tpu_sc as plsc

_ALPHA = 0.5
_NUM_CLASSES = 8192
_FEAT = 32
_BATCH = 16384

_NC = 2
_NS = 16
_NW = _NC * _NS
_RPT = _BATCH // _NW
_CHUNK = 128
_NCHUNK = _RPT // _CHUNK
_CNTW = 16

_ROW_BLK = 512

_PARTS = 2
_PROWS = _BATCH // _PARTS
_RPP = _PROWS // _NW
_NCH_P = _RPP // _CHUNK


def _labels_body(x0_ref, x1_ref, rhs_ref, res_ref, lab_ref):
    acc = jnp.dot(x1_ref[...].astype(jnp.bfloat16), rhs_ref[...],
                  preferred_element_type=jnp.float32)
    g = acc[:, 0:_FEAT]
    lab = acc[:, _FEAT:_FEAT + 1] * 64.0 + acc[:, _FEAT + 1:_FEAT + 2]
    d = x0_ref[...] - g
    res_ref[...] = jnp.sqrt(jnp.sum(d * d, axis=1, keepdims=True))
    lab_ref[...] = lab.astype(jnp.int32)


def _make_stage1(part):
    off = part * (_PROWS // _ROW_BLK)
    return pl.pallas_call(
        _labels_body,
        grid=(_PROWS // _ROW_BLK,),
        in_specs=[
            pl.BlockSpec((_ROW_BLK, _FEAT), lambda i: (i + off, 0)),
            pl.BlockSpec((_ROW_BLK, _NUM_CLASSES), lambda i: (i + off, 0)),
            pl.BlockSpec((_NUM_CLASSES, _FEAT + 2), lambda i: (0, 0)),
        ],
        out_specs=(
            pl.BlockSpec((_ROW_BLK, 1), lambda i: (i, 0)),
            pl.BlockSpec((_ROW_BLK, 1), lambda i: (i, 0)),
        ),
        out_shape=(
            jax.ShapeDtypeStruct((_PROWS, 1), jnp.float32),
            jax.ShapeDtypeStruct((_PROWS, 1), jnp.int32),
        ),
    )


_stage1_calls = [_make_stage1(p) for p in range(_PARTS)]


def _sc_body(part, lab_ref, x0_ref, z32_ref, z16_ref, ones_ref,
             xsum_out, cnt_out,
             xsum_sh, cnt_sh, idx_v, x0_v, ones_v):
    cid = lax.axis_index("c")
    sid = lax.axis_index("s")
    wid = sid * _NC + cid
    base = part * _PROWS + wid * _RPP

    pltpu.sync_copy(z32_ref, xsum_sh.at[pl.ds(sid * _RPT, _RPT), :])
    pltpu.sync_copy(z16_ref, cnt_sh.at[pl.ds(sid * _RPT, _RPT), :])

    pltpu.sync_copy(lab_ref.at[pl.ds(wid * _NCH_P, _NCH_P), :], idx_v)
    pltpu.sync_copy(x0_ref.at[pl.ds(base, _RPP), :], x0_v)
    pltpu.sync_copy(ones_ref, ones_v)

    plsc.subcore_barrier()

    for j in range(_NCH_P):
        pltpu.sync_copy(x0_v.at[pl.ds(j * _CHUNK, _CHUNK), :],
                        xsum_sh.at[idx_v.at[j]], add=True)
        pltpu.sync_copy(ones_v, cnt_sh.at[idx_v.at[j]], add=True)

    plsc.subcore_barrier()

    pltpu.sync_copy(xsum_sh.at[pl.ds(sid * _RPT, _RPT), :],
                    xsum_out.at[cid, pl.ds(sid * _RPT, _RPT), :])
    pltpu.sync_copy(cnt_sh.at[pl.ds(sid * _RPT, _RPT), :],
                    cnt_out.at[cid, pl.ds(sid * _RPT, _RPT), :])


import functools


@functools.cache
def _get_sc_call(part):
  return pl.kernel(
    functools.partial(_sc_body, part),
    out_type=(
        jax.ShapeDtypeStruct((_NC, _NUM_CLASSES, _FEAT), jnp.float32),
        jax.ShapeDtypeStruct((_NC, _NUM_CLASSES, _CNTW), jnp.float32),
    ),
    mesh=plsc.VectorSubcoreMesh(
        core_axis_name="c", subcore_axis_name="s",
        num_cores=_NC, num_subcores=_NS,
    ),
    scratch_types=(
        pltpu.VMEM_SHARED((_NUM_CLASSES, _FEAT), jnp.float32),
        pltpu.VMEM_SHARED((_NUM_CLASSES, _CNTW), jnp.float32),
        pltpu.VMEM((_NCH_P, _CHUNK), jnp.int32),
        pltpu.VMEM((_RPP, _FEAT), jnp.float32),
        pltpu.VMEM((_CHUNK, _CNTW), jnp.float32),
    ),
    compiler_params=pltpu.CompilerParams(use_tc_tiling_on_sc=False),
  )


def _finalize_body(xs0_ref, cnt0_ref, xs1_ref, cnt1_ref, cen_ref, newc_ref):
    n = (cnt0_ref[0, :, 0:1] + cnt0_ref[1, :, 0:1]
         + cnt1_ref[0, :, 0:1] + cnt1_ref[1, :, 0:1])
    xs = (xs0_ref[0] + xs0_ref[1]) + (xs1_ref[0] + xs1_ref[1])
    cen = cen_ref[...]
    delta = (n * cen - xs) / (n + 1.0)
    newc_ref[...] = cen - _ALPHA * delta


_finalize_call = pl.pallas_call(
    _finalize_body,
    out_shape=jax.ShapeDtypeStruct((_NUM_CLASSES, _FEAT), jnp.float32),
)


def kernel(x0, x1, centers):
    hi = (jnp.arange(_NUM_CLASSES, dtype=jnp.int32) // 64).astype(jnp.bfloat16)
    lo = (jnp.arange(_NUM_CLASSES, dtype=jnp.int32) % 64).astype(jnp.bfloat16)
    rhs = jnp.concatenate(
        [centers.astype(jnp.bfloat16), hi[:, None], lo[:, None]], axis=1)
    z32 = jnp.zeros((_RPT, _FEAT), jnp.float32)
    z16 = jnp.zeros((_RPT, _CNTW), jnp.float32)
    ones16 = jnp.zeros((_CHUNK, _CNTW), jnp.float32).at[:, 0].set(1.0)

    results, sc_outs = [], []
    for p in range(_PARTS):
        res_p, lab_p = _stage1_calls[p](x0, x1, rhs)
        results.append(res_p)
        labels_p = lab_p.reshape(_PROWS // _CHUNK, _CHUNK)
        sc_outs.append(_get_sc_call(p)(labels_p, x0, z32, z16, ones16))

    (xs0, c0), (xs1, c1) = sc_outs
    new_centers = _finalize_call(xs0, c0, xs1, c1, centers)
    result = jnp.concatenate(results, axis=0)
    return (result, new_centers)


# speedup vs baseline: 1.1654x; 1.1654x over previous
import jax
import jax.numpy as jnp
from jax import lax
from jax.experimental import pallas as pl
from jax.experimental.pallas import tpu as pltpu
from jax.experimental.pallas import tpu_sc as plsc

_ALPHA = 0.5
_NUM_CLASSES = 8192
_FEAT = 32
_BATCH = 16384

_NC = 2
_NS = 16
_NW = _NC * _NS
_RPT = _BATCH // _NW
_CHUNK = 128
_NCHUNK = _RPT // _CHUNK
_CNTW = 16

_ROW_BLK = 512


def _labels_body(x0_ref, x1_ref, rhs_ref, res_ref, lab_ref, cnt_ref, cnt_acc):
    i = pl.program_id(0)
    blk = x1_ref[...]
    acc = jnp.dot(blk.astype(jnp.bfloat16), rhs_ref[...],
                  preferred_element_type=jnp.float32)
    g = acc[:, 0:_FEAT]
    lab = acc[:, _FEAT:_FEAT + 1] * 64.0 + acc[:, _FEAT + 1:_FEAT + 2]
    d = x0_ref[...] - g
    res_ref[...] = jnp.sqrt(jnp.sum(d * d, axis=1, keepdims=True))
    lab_ref[...] = lab.astype(jnp.int32)

    csum = jnp.sum(blk, axis=0, keepdims=True)

    @pl.when(i == 0)
    def _():
        cnt_acc[...] = jnp.zeros_like(cnt_acc)

    cnt_acc[...] += csum

    @pl.when(i == _BATCH // _ROW_BLK - 1)
    def _():
        cnt_ref[...] = cnt_acc[...]


_labels_call = pl.pallas_call(
    _labels_body,
    grid=(_BATCH // _ROW_BLK,),
    in_specs=[
        pl.BlockSpec((_ROW_BLK, _FEAT), lambda i: (i, 0)),
        pl.BlockSpec((_ROW_BLK, _NUM_CLASSES), lambda i: (i, 0)),
        pl.BlockSpec((_NUM_CLASSES, _FEAT + 2), lambda i: (0, 0)),
    ],
    out_specs=(
        pl.BlockSpec((_ROW_BLK, 1), lambda i: (i, 0)),
        pl.BlockSpec((_ROW_BLK, 1), lambda i: (i, 0)),
        pl.BlockSpec((1, _NUM_CLASSES), lambda i: (0, 0)),
    ),
    out_shape=(
        jax.ShapeDtypeStruct((_BATCH, 1), jnp.float32),
        jax.ShapeDtypeStruct((_BATCH, 1), jnp.int32),
        jax.ShapeDtypeStruct((1, _NUM_CLASSES), jnp.float32),
    ),
    scratch_shapes=[pltpu.VMEM((1, _NUM_CLASSES), jnp.float32)],
)


def _sc_body(lab_ref, x0_ref, z32_ref,
             xsum_out,
             xsum_sh, idx_v, x0_v):
    cid = lax.axis_index("c")
    sid = lax.axis_index("s")
    wid = sid * _NC + cid
    base = wid * _RPT

    pltpu.sync_copy(z32_ref, xsum_sh.at[pl.ds(sid * _RPT, _RPT), :])

    pltpu.sync_copy(lab_ref.at[pl.ds(wid * _NCHUNK, _NCHUNK), :], idx_v)
    pltpu.sync_copy(x0_ref.at[pl.ds(base, _RPT), :], x0_v)

    plsc.subcore_barrier()

    for j in range(_NCHUNK):
        pltpu.sync_copy(x0_v.at[pl.ds(j * _CHUNK, _CHUNK), :],
                        xsum_sh.at[idx_v.at[j]], add=True)

    plsc.subcore_barrier()

    pltpu.sync_copy(xsum_sh.at[pl.ds(sid * _RPT, _RPT), :],
                    xsum_out.at[cid, pl.ds(sid * _RPT, _RPT), :])


import functools


@functools.cache
def _get_sc_call():
  return pl.kernel(
    _sc_body,
    out_type=jax.ShapeDtypeStruct((_NC, _NUM_CLASSES, _FEAT), jnp.float32),
    mesh=plsc.VectorSubcoreMesh(
        core_axis_name="c", subcore_axis_name="s",
        num_cores=_NC, num_subcores=_NS,
    ),
    scratch_types=(
        pltpu.VMEM_SHARED((_NUM_CLASSES, _FEAT), jnp.float32),
        pltpu.VMEM((_NCHUNK, _CHUNK), jnp.int32),
        pltpu.VMEM((_RPT, _FEAT), jnp.float32),
    ),
    compiler_params=pltpu.CompilerParams(use_tc_tiling_on_sc=False),
  )


def _finalize_body(xsum_ref, cnt_ref, cen_ref, newc_ref):
    n = jnp.reshape(cnt_ref[...], (_NUM_CLASSES, 1))
    xs = xsum_ref[0] + xsum_ref[1]
    cen = cen_ref[...]
    delta = (n * cen - xs) / (n + 1.0)
    newc_ref[...] = cen - _ALPHA * delta


_finalize_call = pl.pallas_call(
    _finalize_body,
    out_shape=jax.ShapeDtypeStruct((_NUM_CLASSES, _FEAT), jnp.float32),
)


def kernel(x0, x1, centers):
    hi = (jnp.arange(_NUM_CLASSES, dtype=jnp.int32) // 64).astype(jnp.bfloat16)
    lo = (jnp.arange(_NUM_CLASSES, dtype=jnp.int32) % 64).astype(jnp.bfloat16)
    rhs = jnp.concatenate(
        [centers.astype(jnp.bfloat16), hi[:, None], lo[:, None]], axis=1)
    result, labels, cnt = _labels_call(x0, x1, rhs)
    labels = labels.reshape(_BATCH // _CHUNK, _CHUNK)
    z32 = jnp.zeros((_RPT, _FEAT), jnp.float32)
    xsum = _get_sc_call()(labels, x0, z32)
    new_centers = _finalize_call(xsum, cnt, centers)
    return (result, new_centers)
